# initial kernel scaffold (unmeasured)
import jax
import jax.numpy as jnp
from jax import lax
from jax.experimental import pallas as pl
from jax.experimental.pallas import tpu as pltpu

N_DEV = 8
N_TOK = 2048
D_MODEL = 512
D_HID = 1024
E_LOCAL = 8
CHUNK = N_TOK // N_DEV


def kernel(x, router_W, route_idx, expert_W):
    def body(x_ref, rw_ref, idx_ref, ew_ref, out_ref,
             partial_ref, comm_ref, send_sems, recv_sems):
        my = lax.axis_index("i")
        left = lax.rem(my - 1 + N_DEV, N_DEV)
        right = lax.rem(my + 1, N_DEV)

        barrier_sem = pltpu.get_barrier_semaphore()
        for nbr in (left, right):
            pl.semaphore_signal(
                barrier_sem, inc=1,
                device_id=(nbr,), device_id_type=pl.DeviceIdType.MESH,
            )
        pl.semaphore_wait(barrier_sem, 2)

        xv = x_ref[:, :]
        scores = jnp.dot(xv, rw_ref[:, :], preferred_element_type=jnp.float32)
        smax = jnp.max(scores, axis=-1, keepdims=True)
        pexp = jnp.exp(scores - smax)
        probs = pexp / jnp.sum(pexp, axis=-1, keepdims=True)

        e0 = idx_ref[:, 0:1]
        e1 = idx_ref[:, 1:2]
        eids = lax.broadcasted_iota(jnp.int32, (1, 64), 1)
        g0 = jnp.sum(jnp.where(e0 == eids, probs, 0.0), axis=-1, keepdims=True)
        g1 = jnp.sum(jnp.where(e1 == eids, probs, 0.0), axis=-1, keepdims=True)
        gsum = g0 + g1

        for el in range(E_LOCAL):
            ge = my * E_LOCAL + el
            w = (jnp.where(e0 == ge, g0 / gsum, 0.0)
                 + jnp.where(e1 == ge, g1 / gsum, 0.0))
            contrib = jnp.dot(xv * w, ew_ref[el],
                              preferred_element_type=jnp.float32)
            if el == 0:
                partial_ref[:, :] = contrib
            else:
                partial_ref[:, :] = partial_ref[:, :] + contrib

        c0 = lax.rem(my - 1 + N_DEV, N_DEV)
        comm_ref[0, :, :] = partial_ref[pl.ds(c0 * CHUNK, CHUNK), :]
        for s in range(N_DEV - 1):
            rdma = pltpu.make_async_remote_copy(
                src_ref=comm_ref.at[s],
                dst_ref=comm_ref.at[s + 1],
                send_sem=send_sems.at[s],
                recv_sem=recv_sems.at[s],
                device_id=(right,),
                device_id_type=pl.DeviceIdType.MESH,
            )
            rdma.start()
            rdma.wait()
            c = lax.rem(my - 2 - s + 2 * N_DEV, N_DEV)
            comm_ref[s + 1, :, :] = (
                comm_ref[s + 1, :, :]
                + partial_ref[pl.ds(c * CHUNK, CHUNK), :]
            )
        out_ref[:, :] = comm_ref[N_DEV - 1, :, :]

    return pl.pallas_call(
        body,
        out_shape=jax.ShapeDtypeStruct((CHUNK, D_HID), jnp.float32),
        in_specs=[
            pl.BlockSpec(memory_space=pltpu.VMEM),
            pl.BlockSpec(memory_space=pltpu.VMEM),
            pl.BlockSpec(memory_space=pltpu.VMEM),
            pl.BlockSpec(memory_space=pltpu.VMEM),
        ],
        out_specs=pl.BlockSpec(memory_space=pltpu.VMEM),
        scratch_shapes=[
            pltpu.VMEM((N_TOK, D_HID), jnp.float32),
            pltpu.VMEM((N_DEV, CHUNK, D_HID), jnp.float32),
            pltpu.SemaphoreType.DMA((N_DEV,)),
            pltpu.SemaphoreType.DMA((N_DEV,)),
        ],
        compiler_params=pltpu.CompilerParams(collective_id=0),
    )(x, router_W, route_idx, expert_W)


# baseline (device time: 130595 ns/iter reference)
import jax
import jax.numpy as jnp
from jax import lax
from jax.experimental import pallas as pl
from jax.experimental.pallas import tpu as pltpu

N_DEV = 8
N_TOK = 2048
D_MODEL = 512
D_HID = 1024
E_LOCAL = 8
CHUNK = N_TOK // N_DEV


def kernel(x, router_W, route_idx, expert_W):
    def body(x_ref, rw_ref, idx_ref, ew_ref, out_ref,
             partial_ref, comm_ref, ew_buf, ew_sems, send_sems, recv_sems):
        my = lax.axis_index("i")
        left = lax.rem(my - 1 + N_DEV, N_DEV)
        right = lax.rem(my + 1, N_DEV)

        barrier_sem = pltpu.get_barrier_semaphore()
        for nbr in (left, right):
            pl.semaphore_signal(
                barrier_sem, inc=1,
                device_id=(nbr,), device_id_type=pl.DeviceIdType.MESH,
            )
        pl.semaphore_wait(barrier_sem, 2)

        xv = x_ref[:, :]
        scores = jnp.dot(xv, rw_ref[:, :], preferred_element_type=jnp.float32)
        smax = jnp.max(scores, axis=-1, keepdims=True)
        pexp = jnp.exp(scores - smax)
        probs = pexp / jnp.sum(pexp, axis=-1, keepdims=True)

        e0 = idx_ref[:, 0:1]
        e1 = idx_ref[:, 1:2]
        eids = lax.broadcasted_iota(jnp.int32, (1, 64), 1)
        g0 = jnp.sum(jnp.where(e0 == eids, probs, 0.0), axis=-1, keepdims=True)
        g1 = jnp.sum(jnp.where(e1 == eids, probs, 0.0), axis=-1, keepdims=True)
        gsum = g0 + g1

        pltpu.make_async_copy(ew_ref.at[0], ew_buf.at[0], ew_sems.at[0]).start()
        for el in range(E_LOCAL):
            if el + 1 < E_LOCAL:
                pltpu.make_async_copy(
                    ew_ref.at[el + 1], ew_buf.at[(el + 1) % 2],
                    ew_sems.at[(el + 1) % 2],
                ).start()
            pltpu.make_async_copy(
                ew_ref.at[el], ew_buf.at[el % 2], ew_sems.at[el % 2]
            ).wait()
            ge = my * E_LOCAL + el
            w = (jnp.where(e0 == ge, g0 / gsum, 0.0)
                 + jnp.where(e1 == ge, g1 / gsum, 0.0))
            contrib = jnp.dot(xv * w, ew_buf[el % 2],
                              preferred_element_type=jnp.float32)
            if el == 0:
                partial_ref[:, :] = contrib
            else:
                partial_ref[:, :] = partial_ref[:, :] + contrib

        c0 = lax.rem(my - 1 + N_DEV, N_DEV)
        comm_ref[0, :, :] = partial_ref[pl.ds(c0 * CHUNK, CHUNK), :]
        for s in range(N_DEV - 1):
            rdma = pltpu.make_async_remote_copy(
                src_ref=comm_ref.at[s],
                dst_ref=comm_ref.at[s + 1],
                send_sem=send_sems.at[s],
                recv_sem=recv_sems.at[s],
                device_id=(right,),
                device_id_type=pl.DeviceIdType.MESH,
            )
            rdma.start()
            rdma.wait()
            c = lax.rem(my - 2 - s + 2 * N_DEV, N_DEV)
            comm_ref[s + 1, :, :] = (
                comm_ref[s + 1, :, :]
                + partial_ref[pl.ds(c * CHUNK, CHUNK), :]
            )
        out_ref[:, :] = comm_ref[N_DEV - 1, :, :]

    return pl.pallas_call(
        body,
        out_shape=jax.ShapeDtypeStruct((CHUNK, D_HID), jnp.float32),
        in_specs=[
            pl.BlockSpec(memory_space=pltpu.VMEM),
            pl.BlockSpec(memory_space=pltpu.VMEM),
            pl.BlockSpec(memory_space=pltpu.VMEM),
            pl.BlockSpec(memory_space=pl.ANY),
        ],
        out_specs=pl.BlockSpec(memory_space=pltpu.VMEM),
        scratch_shapes=[
            pltpu.VMEM((N_TOK, D_HID), jnp.float32),
            pltpu.VMEM((N_DEV, CHUNK, D_HID), jnp.float32),
            pltpu.VMEM((2, D_MODEL, D_HID), jnp.float32),
            pltpu.SemaphoreType.DMA((2,)),
            pltpu.SemaphoreType.DMA((N_DEV,)),
            pltpu.SemaphoreType.DMA((N_DEV,)),
        ],
        compiler_params=pltpu.CompilerParams(
            collective_id=0,
            vmem_limit_bytes=64 * 1024 * 1024,
        ),
    )(x, router_W, route_idx, expert_W)


# device time: 122630 ns/iter; 1.0650x vs baseline; 1.0650x over previous
import jax
import jax.numpy as jnp
from jax import lax
from jax.experimental import pallas as pl
from jax.experimental.pallas import tpu as pltpu

N_DEV = 8
N_TOK = 2048
D_MODEL = 512
D_HID = 1024
E_LOCAL = 8
CHUNK = N_TOK // N_DEV


def kernel(x, router_W, route_idx, expert_W):
    def body(x_ref, rw_ref, idx_ref, ew_ref, out_ref,
             w_ref, comm_ref, send_sems, recv_sems):
        my = lax.axis_index("i")
        left = lax.rem(my - 1 + N_DEV, N_DEV)
        right = lax.rem(my + 1, N_DEV)

        barrier_sem = pltpu.get_barrier_semaphore()
        for nbr in (left, right):
            pl.semaphore_signal(
                barrier_sem, inc=1,
                device_id=(nbr,), device_id_type=pl.DeviceIdType.MESH,
            )
        pl.semaphore_wait(barrier_sem, 2)

        xv = x_ref[:, :]
        scores = jnp.dot(xv, rw_ref[:, :], preferred_element_type=jnp.float32)
        smax = jnp.max(scores, axis=-1, keepdims=True)
        pexp = jnp.exp(scores - smax)
        probs = pexp / jnp.sum(pexp, axis=-1, keepdims=True)

        e0 = idx_ref[:, 0:1]
        e1 = idx_ref[:, 1:2]
        eids = lax.broadcasted_iota(jnp.int32, (1, 64), 1)
        g0 = jnp.sum(jnp.where(e0 == eids, probs, 0.0), axis=-1, keepdims=True)
        g1 = jnp.sum(jnp.where(e1 == eids, probs, 0.0), axis=-1, keepdims=True)
        gsum = g0 + g1

        cols = []
        for el in range(E_LOCAL):
            ge = my * E_LOCAL + el
            cols.append(jnp.where(e0 == ge, g0 / gsum, 0.0)
                        + jnp.where(e1 == ge, g1 / gsum, 0.0))
        w_ref[:, :] = jnp.concatenate(cols, axis=1)

        def chunk_partial(c):
            row0 = c * CHUNK
            xc = x_ref[pl.ds(row0, CHUNK), :]
            wc = w_ref[pl.ds(row0, CHUNK), :]
            acc = None
            for el in range(E_LOCAL):
                t = jnp.dot(xc * wc[:, el:el + 1], ew_ref[el],
                            preferred_element_type=jnp.float32)
                acc = t if acc is None else acc + t
            return acc

        c0 = lax.rem(my - 1 + N_DEV, N_DEV)
        comm_ref[0, :, :] = chunk_partial(c0)
        for s in range(N_DEV - 1):
            rdma = pltpu.make_async_remote_copy(
                src_ref=comm_ref.at[s],
                dst_ref=comm_ref.at[s + 1],
                send_sem=send_sems.at[s],
                recv_sem=recv_sems.at[s],
                device_id=(right,),
                device_id_type=pl.DeviceIdType.MESH,
            )
            rdma.start()
            c = lax.rem(my - 2 - s + 2 * N_DEV, N_DEV)
            tmp = chunk_partial(c)
            rdma.wait()
            comm_ref[s + 1, :, :] = comm_ref[s + 1, :, :] + tmp
        out_ref[:, :] = comm_ref[N_DEV - 1, :, :]

    return pl.pallas_call(
        body,
        out_shape=jax.ShapeDtypeStruct((CHUNK, D_HID), jnp.float32),
        in_specs=[
            pl.BlockSpec(memory_space=pltpu.VMEM),
            pl.BlockSpec(memory_space=pltpu.VMEM),
            pl.BlockSpec(memory_space=pltpu.VMEM),
            pl.BlockSpec(memory_space=pltpu.VMEM),
        ],
        out_specs=pl.BlockSpec(memory_space=pltpu.VMEM),
        scratch_shapes=[
            pltpu.VMEM((N_TOK, E_LOCAL), jnp.float32),
            pltpu.VMEM((N_DEV, CHUNK, D_HID), jnp.float32),
            pltpu.SemaphoreType.DMA((N_DEV,)),
            pltpu.SemaphoreType.DMA((N_DEV,)),
        ],
        compiler_params=pltpu.CompilerParams(
            collective_id=0,
            vmem_limit_bytes=64 * 1024 * 1024,
        ),
    )(x, router_W, route_idx, expert_W)


# device time: 119312 ns/iter; 1.0946x vs baseline; 1.0278x over previous
import jax
import jax.numpy as jnp
from jax import lax
from jax.experimental import pallas as pl
from jax.experimental.pallas import tpu as pltpu

N_DEV = 8
N_TOK = 2048
D_MODEL = 512
D_HID = 1024
E_LOCAL = 8
CHUNK = N_TOK // N_DEV


def kernel(x, router_W, route_idx, expert_W):
    def body(x_ref, rw_ref, idx_ref, ew_ref, out_ref,
             w_ref, ew_bf, comm_ref, send_sems, recv_sems):
        my = lax.axis_index("i")
        left = lax.rem(my - 1 + N_DEV, N_DEV)
        right = lax.rem(my + 1, N_DEV)

        barrier_sem = pltpu.get_barrier_semaphore()
        for nbr in (left, right):
            pl.semaphore_signal(
                barrier_sem, inc=1,
                device_id=(nbr,), device_id_type=pl.DeviceIdType.MESH,
            )
        pl.semaphore_wait(barrier_sem, 2)

        xv = x_ref[:, :]
        scores = jnp.dot(xv, rw_ref[:, :], preferred_element_type=jnp.float32)
        smax = jnp.max(scores, axis=-1, keepdims=True)
        pexp = jnp.exp(scores - smax)
        probs = pexp / jnp.sum(pexp, axis=-1, keepdims=True)

        e0 = idx_ref[:, 0:1]
        e1 = idx_ref[:, 1:2]
        eids = lax.broadcasted_iota(jnp.int32, (1, 64), 1)
        g0 = jnp.sum(jnp.where(e0 == eids, probs, 0.0), axis=-1, keepdims=True)
        g1 = jnp.sum(jnp.where(e1 == eids, probs, 0.0), axis=-1, keepdims=True)
        gsum = g0 + g1

        cols = []
        for el in range(E_LOCAL):
            ge = my * E_LOCAL + el
            cols.append(jnp.where(e0 == ge, g0 / gsum, 0.0)
                        + jnp.where(e1 == ge, g1 / gsum, 0.0))
        w_ref[:, :] = jnp.concatenate(cols, axis=1)

        ew_bf[:, :, :] = ew_ref[:, :, :].astype(jnp.bfloat16)

        def chunk_partial(c):
            row0 = c * CHUNK
            xc = x_ref[pl.ds(row0, CHUNK), :].astype(jnp.bfloat16)
            wc = w_ref[pl.ds(row0, CHUNK), :]
            acc = None
            for el in range(E_LOCAL):
                t = jnp.dot(xc, ew_bf[el],
                            preferred_element_type=jnp.float32) * wc[:, el:el + 1]
                acc = t if acc is None else acc + t
            return acc

        c0 = lax.rem(my - 1 + N_DEV, N_DEV)
        comm_ref[0, :, :] = chunk_partial(c0)
        for s in range(N_DEV - 1):
            rdma = pltpu.make_async_remote_copy(
                src_ref=comm_ref.at[s],
                dst_ref=comm_ref.at[s + 1],
                send_sem=send_sems.at[s],
                recv_sem=recv_sems.at[s],
                device_id=(right,),
                device_id_type=pl.DeviceIdType.MESH,
            )
            rdma.start()
            c = lax.rem(my - 2 - s + 2 * N_DEV, N_DEV)
            tmp = chunk_partial(c)
            rdma.wait()
            comm_ref[s + 1, :, :] = comm_ref[s + 1, :, :] + tmp
        out_ref[:, :] = comm_ref[N_DEV - 1, :, :]

    return pl.pallas_call(
        body,
        out_shape=jax.ShapeDtypeStruct((CHUNK, D_HID), jnp.float32),
        in_specs=[
            pl.BlockSpec(memory_space=pltpu.VMEM),
            pl.BlockSpec(memory_space=pltpu.VMEM),
            pl.BlockSpec(memory_space=pltpu.VMEM),
            pl.BlockSpec(memory_space=pltpu.VMEM),
        ],
        out_specs=pl.BlockSpec(memory_space=pltpu.VMEM),
        scratch_shapes=[
            pltpu.VMEM((N_TOK, E_LOCAL), jnp.float32),
            pltpu.VMEM((E_LOCAL, D_MODEL, D_HID), jnp.bfloat16),
            pltpu.VMEM((N_DEV, CHUNK, D_HID), jnp.float32),
            pltpu.SemaphoreType.DMA((N_DEV,)),
            pltpu.SemaphoreType.DMA((N_DEV,)),
        ],
        compiler_params=pltpu.CompilerParams(
            collective_id=0,
            vmem_limit_bytes=64 * 1024 * 1024,
        ),
    )(x, router_W, route_idx, expert_W)


# device time: 116550 ns/iter; 1.1205x vs baseline; 1.0237x over previous
import os

import jax
import jax.numpy as jnp
from jax import lax
from jax.experimental import pallas as pl
from jax.experimental.pallas import tpu as pltpu

try:
    with open(os.path.join(os.path.dirname(__file__), "diag_flags.txt")) as _f:
        _FLAGS = _f.read().split()
except OSError:
    _FLAGS = []
_SKIP_RING = "skip_ring" in _FLAGS
_SKIP_COMPUTE = "skip_compute" in _FLAGS

N_DEV = 8
N_TOK = 2048
D_MODEL = 512
D_HID = 1024
E_LOCAL = 8
CHUNK = N_TOK // N_DEV


def kernel(x, router_W, route_idx, expert_W):
    def body(x_ref, rw_ref, idx_ref, ew_ref, out_ref,
             w_ref, ew_bf, comm_ref, send_sems, recv_sems):
        my = lax.axis_index("i")
        left = lax.rem(my - 1 + N_DEV, N_DEV)
        right = lax.rem(my + 1, N_DEV)

        barrier_sem = pltpu.get_barrier_semaphore()
        for nbr in (left, right):
            pl.semaphore_signal(
                barrier_sem, inc=1,
                device_id=(nbr,), device_id_type=pl.DeviceIdType.MESH,
            )
        pl.semaphore_wait(barrier_sem, 2)

        xv = x_ref[:, :]
        scores = jnp.dot(xv, rw_ref[:, :], preferred_element_type=jnp.float32)
        smax = jnp.max(scores, axis=-1, keepdims=True)
        pexp = jnp.exp(scores - smax)
        probs = pexp / jnp.sum(pexp, axis=-1, keepdims=True)

        e0 = idx_ref[:, 0:1]
        e1 = idx_ref[:, 1:2]
        eids = lax.broadcasted_iota(jnp.int32, (1, 64), 1)
        g0 = jnp.sum(jnp.where(e0 == eids, probs, 0.0), axis=-1, keepdims=True)
        g1 = jnp.sum(jnp.where(e1 == eids, probs, 0.0), axis=-1, keepdims=True)
        gsum = g0 + g1

        cols = []
        for el in range(E_LOCAL):
            ge = my * E_LOCAL + el
            cols.append(jnp.where(e0 == ge, g0 / gsum, 0.0)
                        + jnp.where(e1 == ge, g1 / gsum, 0.0))
        w_ref[:, :] = jnp.concatenate(cols, axis=1)

        ew_bf[:, :, :] = ew_ref[:, :, :].astype(jnp.bfloat16)

        def chunk_partial(c):
            row0 = c * CHUNK
            if _SKIP_COMPUTE:
                return jnp.zeros((CHUNK, D_HID), jnp.float32)
            xc = x_ref[pl.ds(row0, CHUNK), :].astype(jnp.bfloat16)
            wc = w_ref[pl.ds(row0, CHUNK), :]
            acc = None
            for el in range(E_LOCAL):
                t = jnp.dot(xc, ew_bf[el],
                            preferred_element_type=jnp.float32) * wc[:, el:el + 1]
                acc = t if acc is None else acc + t
            return acc

        c0 = lax.rem(my - 1 + N_DEV, N_DEV)
        comm_ref[0, :, :] = chunk_partial(c0)
        for s in range(N_DEV - 1):
            rdma = pltpu.make_async_remote_copy(
                src_ref=comm_ref.at[s],
                dst_ref=comm_ref.at[s + 1],
                send_sem=send_sems.at[s],
                recv_sem=recv_sems.at[s],
                device_id=(right,),
                device_id_type=pl.DeviceIdType.MESH,
            )
            if not _SKIP_RING:
                rdma.start()
            c = lax.rem(my - 2 - s + 2 * N_DEV, N_DEV)
            tmp = chunk_partial(c)
            if not _SKIP_RING:
                rdma.wait()
            comm_ref[s + 1, :, :] = comm_ref[s + 1, :, :] + tmp
        out_ref[:, :] = comm_ref[N_DEV - 1, :, :]

    return pl.pallas_call(
        body,
        out_shape=jax.ShapeDtypeStruct((CHUNK, D_HID), jnp.float32),
        in_specs=[
            pl.BlockSpec(memory_space=pltpu.VMEM),
            pl.BlockSpec(memory_space=pltpu.VMEM),
            pl.BlockSpec(memory_space=pltpu.VMEM),
            pl.BlockSpec(memory_space=pltpu.VMEM),
        ],
        out_specs=pl.BlockSpec(memory_space=pltpu.VMEM),
        scratch_shapes=[
            pltpu.VMEM((N_TOK, E_LOCAL), jnp.float32),
            pltpu.VMEM((E_LOCAL, D_MODEL, D_HID), jnp.bfloat16),
            pltpu.VMEM((N_DEV, CHUNK, D_HID), jnp.float32),
            pltpu.SemaphoreType.DMA((N_DEV,)),
            pltpu.SemaphoreType.DMA((N_DEV,)),
        ],
        compiler_params=pltpu.CompilerParams(
            collective_id=0,
            vmem_limit_bytes=64 * 1024 * 1024,
        ),
    )(x, router_W, route_idx, expert_W)


# device time: 63517 ns/iter; 2.0561x vs baseline; 1.8349x over previous
import jax
import jax.numpy as jnp
from jax import lax
from jax.experimental import pallas as pl
from jax.experimental.pallas import tpu as pltpu

N_DEV = 8
N_TOK = 2048
D_MODEL = 512
D_HID = 1024
E_LOCAL = 8
CHUNK = N_TOK // N_DEV
HALF = CHUNK // 2


def kernel(x, router_W, route_idx, expert_W):
    def body(x_ref, rw_ref, idx_ref, ew_ref, out_ref,
             w_ref, ew_bf, commR, commL,
             sendR, recvR, sendL, recvL):
        my = lax.axis_index("i")
        left = lax.rem(my - 1 + N_DEV, N_DEV)
        right = lax.rem(my + 1, N_DEV)

        barrier_sem = pltpu.get_barrier_semaphore()
        for nbr in (left, right):
            pl.semaphore_signal(
                barrier_sem, inc=1,
                device_id=(nbr,), device_id_type=pl.DeviceIdType.MESH,
            )
        pl.semaphore_wait(barrier_sem, 2)

        xv = x_ref[:, :]
        scores = jnp.dot(xv, rw_ref[:, :], preferred_element_type=jnp.float32)
        smax = jnp.max(scores, axis=-1, keepdims=True)
        pexp = jnp.exp(scores - smax)
        probs = pexp / jnp.sum(pexp, axis=-1, keepdims=True)

        e0 = idx_ref[:, 0:1]
        e1 = idx_ref[:, 1:2]
        eids = lax.broadcasted_iota(jnp.int32, (1, 64), 1)
        g0 = jnp.sum(jnp.where(e0 == eids, probs, 0.0), axis=-1, keepdims=True)
        g1 = jnp.sum(jnp.where(e1 == eids, probs, 0.0), axis=-1, keepdims=True)
        gsum = g0 + g1

        cols = []
        for el in range(E_LOCAL):
            ge = my * E_LOCAL + el
            cols.append(jnp.where(e0 == ge, g0 / gsum, 0.0)
                        + jnp.where(e1 == ge, g1 / gsum, 0.0))
        w_ref[:, :] = jnp.concatenate(cols, axis=1)

        ew_bf[:, :, :] = ew_ref[:, :, :].astype(jnp.bfloat16)

        def half_partial(c, off):
            row0 = c * CHUNK + off
            xc = x_ref[pl.ds(row0, HALF), :].astype(jnp.bfloat16)
            wc = w_ref[pl.ds(row0, HALF), :]
            acc = None
            for el in range(E_LOCAL):
                t = jnp.dot(xc, ew_bf[el],
                            preferred_element_type=jnp.float32) * wc[:, el:el + 1]
                acc = t if acc is None else acc + t
            return acc

        cR0 = lax.rem(my - 1 + N_DEV, N_DEV)
        cL0 = lax.rem(my + 1, N_DEV)
        commR[0, :, :] = half_partial(cR0, 0).astype(jnp.bfloat16)
        commL[0, :, :] = half_partial(cL0, HALF).astype(jnp.bfloat16)
        for s in range(N_DEV - 1):
            rdR = pltpu.make_async_remote_copy(
                src_ref=commR.at[s], dst_ref=commR.at[s + 1],
                send_sem=sendR.at[s], recv_sem=recvR.at[s],
                device_id=(right,), device_id_type=pl.DeviceIdType.MESH,
            )
            rdL = pltpu.make_async_remote_copy(
                src_ref=commL.at[s], dst_ref=commL.at[s + 1],
                send_sem=sendL.at[s], recv_sem=recvL.at[s],
                device_id=(left,), device_id_type=pl.DeviceIdType.MESH,
            )
            rdR.start()
            rdL.start()
            cR = lax.rem(my - 2 - s + 2 * N_DEV, N_DEV)
            cL = lax.rem(my + 2 + s, N_DEV)
            tR = half_partial(cR, 0)
            tL = half_partial(cL, HALF)
            rdR.wait()
            rdL.wait()
            if s < N_DEV - 2:
                commR[s + 1, :, :] = (commR[s + 1, :, :] + tR).astype(jnp.bfloat16)
                commL[s + 1, :, :] = (commL[s + 1, :, :] + tL).astype(jnp.bfloat16)
            else:
                out_ref[0:HALF, :] = commR[s + 1, :, :].astype(jnp.float32) + tR
                out_ref[HALF:CHUNK, :] = commL[s + 1, :, :].astype(jnp.float32) + tL

    return pl.pallas_call(
        body,
        out_shape=jax.ShapeDtypeStruct((CHUNK, D_HID), jnp.float32),
        in_specs=[
            pl.BlockSpec(memory_space=pltpu.VMEM),
            pl.BlockSpec(memory_space=pltpu.VMEM),
            pl.BlockSpec(memory_space=pltpu.VMEM),
            pl.BlockSpec(memory_space=pltpu.VMEM),
        ],
        out_specs=pl.BlockSpec(memory_space=pltpu.VMEM),
        scratch_shapes=[
            pltpu.VMEM((N_TOK, E_LOCAL), jnp.float32),
            pltpu.VMEM((E_LOCAL, D_MODEL, D_HID), jnp.bfloat16),
            pltpu.VMEM((N_DEV, HALF, D_HID), jnp.bfloat16),
            pltpu.VMEM((N_DEV, HALF, D_HID), jnp.bfloat16),
            pltpu.SemaphoreType.DMA((N_DEV,)),
            pltpu.SemaphoreType.DMA((N_DEV,)),
            pltpu.SemaphoreType.DMA((N_DEV,)),
            pltpu.SemaphoreType.DMA((N_DEV,)),
        ],
        compiler_params=pltpu.CompilerParams(
            collective_id=0,
            vmem_limit_bytes=64 * 1024 * 1024,
        ),
    )(x, router_W, route_idx, expert_W)


# device time: 60054 ns/iter; 2.1746x vs baseline; 1.0577x over previous
import os

import jax
import jax.numpy as jnp
from jax import lax
from jax.experimental import pallas as pl
from jax.experimental.pallas import tpu as pltpu

try:
    with open(os.path.join(os.path.dirname(__file__), "diag_flags.txt")) as _f:
        _FLAGS = _f.read().split()
except OSError:
    _FLAGS = []
_SKIP_RING = "skip_ring" in _FLAGS
_SKIP_COMPUTE = "skip_compute" in _FLAGS

N_DEV = 8
N_TOK = 2048
D_MODEL = 512
D_HID = 1024
E_LOCAL = 8
CHUNK = N_TOK // N_DEV


def kernel(x, router_W, route_idx, expert_W):
    def body(x_ref, rw_ref, idx_ref, ew_ref, out_ref,
             w_ref, ew_bf, send_buf, recv_buf, send_sems, recv_sems):
        my = lax.axis_index("i")

        barrier_sem = pltpu.get_barrier_semaphore()
        for k in range(1, N_DEV):
            pl.semaphore_signal(
                barrier_sem, inc=1,
                device_id=(lax.rem(my + k, N_DEV),),
                device_id_type=pl.DeviceIdType.MESH,
            )
        pl.semaphore_wait(barrier_sem, N_DEV - 1)

        xv = x_ref[:, :]
        scores = jnp.dot(xv, rw_ref[:, :], preferred_element_type=jnp.float32)
        smax = jnp.max(scores, axis=-1, keepdims=True)
        pexp = jnp.exp(scores - smax)
        probs = pexp / jnp.sum(pexp, axis=-1, keepdims=True)

        e0 = idx_ref[:, 0:1]
        e1 = idx_ref[:, 1:2]
        eids = lax.broadcasted_iota(jnp.int32, (1, 64), 1)
        g0 = jnp.sum(jnp.where(e0 == eids, probs, 0.0), axis=-1, keepdims=True)
        g1 = jnp.sum(jnp.where(e1 == eids, probs, 0.0), axis=-1, keepdims=True)
        gsum = g0 + g1

        cols = []
        for el in range(E_LOCAL):
            ge = my * E_LOCAL + el
            cols.append(jnp.where(e0 == ge, g0 / gsum, 0.0)
                        + jnp.where(e1 == ge, g1 / gsum, 0.0))
        w_ref[:, :] = jnp.concatenate(cols, axis=1)

        ew_bf[:, :, :] = ew_ref[:, :, :].astype(jnp.bfloat16)

        def chunk_partial(c):
            if _SKIP_COMPUTE:
                return jnp.zeros((CHUNK, D_HID), jnp.float32)
            row0 = c * CHUNK
            xc = x_ref[pl.ds(row0, CHUNK), :].astype(jnp.bfloat16)
            wc = w_ref[pl.ds(row0, CHUNK), :]
            acc = None
            for el in range(E_LOCAL):
                t = jnp.dot(xc, ew_bf[el],
                            preferred_element_type=jnp.float32) * wc[:, el:el + 1]
                acc = t if acc is None else acc + t
            return acc

        rdmas = []
        for k in range(1, N_DEV):
            dest = lax.rem(my - k + N_DEV, N_DEV)
            send_buf[k - 1, :, :] = chunk_partial(dest).astype(jnp.bfloat16)
            rdma = pltpu.make_async_remote_copy(
                src_ref=send_buf.at[k - 1],
                dst_ref=recv_buf.at[k - 1],
                send_sem=send_sems.at[k - 1],
                recv_sem=recv_sems.at[k - 1],
                device_id=(dest,),
                device_id_type=pl.DeviceIdType.MESH,
            )
            if not _SKIP_RING:
                rdma.start()
            rdmas.append(rdma)

        out_ref[:, :] = chunk_partial(my)

        for k in range(1, N_DEV):
            if not _SKIP_RING:
                rdmas[k - 1].wait_recv()
            out_ref[:, :] = out_ref[:, :] + recv_buf[k - 1, :, :].astype(jnp.float32)

        for k in range(1, N_DEV):
            if not _SKIP_RING:
                rdmas[k - 1].wait_send()

    return pl.pallas_call(
        body,
        out_shape=jax.ShapeDtypeStruct((CHUNK, D_HID), jnp.float32),
        in_specs=[
            pl.BlockSpec(memory_space=pltpu.VMEM),
            pl.BlockSpec(memory_space=pltpu.VMEM),
            pl.BlockSpec(memory_space=pltpu.VMEM),
            pl.BlockSpec(memory_space=pltpu.VMEM),
        ],
        out_specs=pl.BlockSpec(memory_space=pltpu.VMEM),
        scratch_shapes=[
            pltpu.VMEM((N_TOK, E_LOCAL), jnp.float32),
            pltpu.VMEM((E_LOCAL, D_MODEL, D_HID), jnp.bfloat16),
            pltpu.VMEM((N_DEV - 1, CHUNK, D_HID), jnp.bfloat16),
            pltpu.VMEM((N_DEV - 1, CHUNK, D_HID), jnp.bfloat16),
            pltpu.SemaphoreType.DMA((N_DEV - 1,)),
            pltpu.SemaphoreType.DMA((N_DEV - 1,)),
        ],
        compiler_params=pltpu.CompilerParams(
            collective_id=0,
            vmem_limit_bytes=64 * 1024 * 1024,
        ),
    )(x, router_W, route_idx, expert_W)
